# baseline (speedup 1.0000x reference)
import functools

import jax
import jax.numpy as jnp
from jax import lax
from jax.experimental import pallas as pl
from jax.experimental.pallas import tpu as pltpu
from jax.experimental.pallas import tpu_sc as plsc

NC = 2
NS = 16
NW = NC * NS
IPD = 128
CHUNK = 512
K = CHUNK // IPD


def _make_gather(V, D, B):
  assert B % (NW * CHUNK) == 0
  rows_per_w = B // NW
  groups = rows_per_w // CHUNK
  mesh = plsc.VectorSubcoreMesh(core_axis_name="c", subcore_axis_name="s")

  @functools.partial(
      pl.kernel,
      mesh=mesh,
      compiler_params=pltpu.CompilerParams(use_tc_tiling_on_sc=False),
      out_type=jax.ShapeDtypeStruct((B, D), jnp.float32),
      scratch_types=[
          pltpu.VMEM((K, IPD), jnp.int32),
          pltpu.VMEM((CHUNK, D), jnp.float32),
          pltpu.SemaphoreType.DMA,
      ],
  )
  def k(table_hbm, idx_hbm, out_hbm, idx_v, rows_v, gsem):
    wid = lax.axis_index("s") * NC + lax.axis_index("c")
    base_irow = wid * (rows_per_w // IPD)

    def group(g, carry):
      irow = base_irow + g * K
      pltpu.sync_copy(idx_hbm.at[pl.ds(irow, K)], idx_v)
      copies = []
      for j in range(K):
        copies.append(
            pltpu.async_copy(
                table_hbm.at[idx_v.at[j]],
                rows_v.at[pl.ds(j * IPD, IPD)],
                gsem,
            ))
      for cp in copies:
        cp.wait()
      pltpu.sync_copy(rows_v, out_hbm.at[pl.ds(irow * IPD, CHUNK)])
      return carry

    lax.fori_loop(0, groups, group, 0)

  return k


def kernel(x, embedding):
  B = x.shape[0] * x.shape[1]
  D = embedding.shape[1]
  idx = x.reshape(B // IPD, IPD).astype(jnp.int32)
  out = _make_gather(embedding.shape[0], D, B)(embedding, idx)
  return out.reshape(x.shape + (D,))


# speedup vs baseline: 1.0447x; 1.0447x over previous
import functools

import jax
import jax.numpy as jnp
from jax import lax
from jax.experimental import pallas as pl
from jax.experimental.pallas import tpu as pltpu
from jax.experimental.pallas import tpu_sc as plsc

NC = 2
NS = 16
NW = NC * NS
IPD = 128
CHUNK = 512
K = CHUNK // IPD


def _make_gather(V, D, B):
  assert B % (NW * CHUNK) == 0
  rows_per_w = B // NW
  G = rows_per_w // CHUNK
  assert G >= 2 and G % 2 == 0
  mesh = plsc.VectorSubcoreMesh(core_axis_name="c", subcore_axis_name="s")

  @functools.partial(
      pl.kernel,
      mesh=mesh,
      compiler_params=pltpu.CompilerParams(use_tc_tiling_on_sc=False),
      out_type=jax.ShapeDtypeStruct((B, D), jnp.float32),
      scratch_types=[
          pltpu.VMEM((2, K, IPD), jnp.int32),
          pltpu.VMEM((2, CHUNK, D), jnp.float32),
          pltpu.SemaphoreType.DMA((2,)),
          pltpu.SemaphoreType.DMA((2,)),
      ],
  )
  def k(table_hbm, idx_hbm, out_hbm, idx_v, rows_v, gsem, ssem):
    wid = lax.axis_index("s") * NC + lax.axis_index("c")
    base_irow = wid * (rows_per_w // IPD)

    def load_idx(g, b):
      pltpu.sync_copy(idx_hbm.at[pl.ds(base_irow + g * K, K)], idx_v.at[b])

    def fire_gathers(b):
      for j in range(K):
        pltpu.async_copy(
            table_hbm.at[idx_v.at[b, j]],
            rows_v.at[b, pl.ds(j * IPD, IPD)],
            gsem.at[b],
        )

    def drain_gathers(b):
      for _ in range(K):
        pltpu.make_async_copy(
            table_hbm.at[idx_v.at[b, 0]],
            rows_v.at[b, pl.ds(0, IPD)],
            gsem.at[b],
        ).wait()

    def store(g, b):
      return pltpu.async_copy(
          rows_v.at[b],
          out_hbm.at[pl.ds((base_irow + g * K) * IPD, CHUNK)],
          ssem.at[b],
      )

    def drain_store(g, b):
      pltpu.make_async_copy(
          rows_v.at[b],
          out_hbm.at[pl.ds((base_irow + g * K) * IPD, CHUNK)],
          ssem.at[b],
      ).wait()

    load_idx(0, 0)
    fire_gathers(0)
    load_idx(1, 1)
    fire_gathers(1)

    def pair(u, carry):
      for b in range(2):
        g = 2 * u + b
        drain_gathers(b)
        store(g, b)
        load_idx(g + 2, b)
        drain_store(g, b)
        fire_gathers(b)
      return carry

    lax.fori_loop(0, (G - 2) // 2, pair, 0)

    for b in range(2):
      g = G - 2 + b
      drain_gathers(b)
      store(g, b)
    for b in range(2):
      drain_store(G - 2 + b, b)

  return k


def kernel(x, embedding):
  B = x.shape[0] * x.shape[1]
  D = embedding.shape[1]
  idx = x.reshape(B // IPD, IPD).astype(jnp.int32)
  out = _make_gather(embedding.shape[0], D, B)(embedding, idx)
  return out.reshape(x.shape + (D,))


# speedup vs baseline: 1.0464x; 1.0016x over previous
import functools

import jax
import jax.numpy as jnp
from jax import lax
from jax.experimental import pallas as pl
from jax.experimental.pallas import tpu as pltpu
from jax.experimental.pallas import tpu_sc as plsc

NC = 2
NS = 16
NW = NC * NS
IPD = 512
CHUNK = 512
K = CHUNK // IPD


def _make_gather(V, D, B):
  assert B % (NW * CHUNK) == 0
  rows_per_w = B // NW
  G = rows_per_w // CHUNK
  assert G >= 2 and G % 2 == 0
  mesh = plsc.VectorSubcoreMesh(core_axis_name="c", subcore_axis_name="s")

  @functools.partial(
      pl.kernel,
      mesh=mesh,
      compiler_params=pltpu.CompilerParams(use_tc_tiling_on_sc=False),
      out_type=jax.ShapeDtypeStruct((B, D), jnp.float32),
      scratch_types=[
          pltpu.VMEM((2, K, IPD), jnp.int32),
          pltpu.VMEM((2, CHUNK, D), jnp.float32),
          pltpu.SemaphoreType.DMA((2,)),
          pltpu.SemaphoreType.DMA((2,)),
      ],
  )
  def k(table_hbm, idx_hbm, out_hbm, idx_v, rows_v, gsem, ssem):
    wid = lax.axis_index("s") * NC + lax.axis_index("c")
    base_irow = wid * (rows_per_w // IPD)

    def load_idx(g, b):
      pltpu.sync_copy(idx_hbm.at[pl.ds(base_irow + g * K, K)], idx_v.at[b])

    def fire_gathers(b):
      for j in range(K):
        pltpu.async_copy(
            table_hbm.at[idx_v.at[b, j]],
            rows_v.at[b, pl.ds(j * IPD, IPD)],
            gsem.at[b],
        )

    def drain_gathers(b):
      for _ in range(K):
        pltpu.make_async_copy(
            table_hbm.at[idx_v.at[b, 0]],
            rows_v.at[b, pl.ds(0, IPD)],
            gsem.at[b],
        ).wait()

    def store(g, b):
      return pltpu.async_copy(
          rows_v.at[b],
          out_hbm.at[pl.ds((base_irow + g * K) * IPD, CHUNK)],
          ssem.at[b],
      )

    def drain_store(g, b):
      pltpu.make_async_copy(
          rows_v.at[b],
          out_hbm.at[pl.ds((base_irow + g * K) * IPD, CHUNK)],
          ssem.at[b],
      ).wait()

    load_idx(0, 0)
    fire_gathers(0)
    load_idx(1, 1)
    fire_gathers(1)

    def pair(u, carry):
      for b in range(2):
        g = 2 * u + b
        drain_gathers(b)
        store(g, b)
        load_idx(g + 2, b)
        drain_store(g, b)
        fire_gathers(b)
      return carry

    lax.fori_loop(0, (G - 2) // 2, pair, 0)

    for b in range(2):
      g = G - 2 + b
      drain_gathers(b)
      store(g, b)
    for b in range(2):
      drain_store(G - 2 + b, b)

  return k


def kernel(x, embedding):
  B = x.shape[0] * x.shape[1]
  D = embedding.shape[1]
  idx = x.reshape(B // IPD, IPD).astype(jnp.int32)
  out = _make_gather(embedding.shape[0], D, B)(embedding, idx)
  return out.reshape(x.shape + (D,))
